# baseline (speedup 1.0000x reference)
import functools

import jax
import jax.numpy as jnp
from jax import lax
from jax.experimental import pallas as pl
from jax.experimental.pallas import tpu as pltpu
from jax.experimental.pallas import tpu_sc as plsc

_N, _E, _FIN, _H, _C = 10000, 320000, 128, 16, 40
_NC, _NS = 2, 16
_EPS = _E // _NS
_B = 100
_CHT = _EPS // _B
_CH2 = _CHT // _NC
_K = 5
_GG1 = _CHT // (2 * _K)
_GG2 = _CH2 // (2 * _K)
_RPT = 1000
_NSTAGE = _N // _RPT
_RPC = _N // _NS
_DW = 16

_SC_PARAMS = pltpu.CompilerParams(use_tc_tiling_on_sc=False)


def _sc_degree(dst3, ones_rows, zeros_d):
    mesh = plsc.VectorSubcoreMesh(core_axis_name="c", subcore_axis_name="s")

    @functools.partial(
        pl.kernel,
        out_type=jax.ShapeDtypeStruct((_NC, _N, _DW), jnp.float32),
        mesh=mesh,
        compiler_params=_SC_PARAMS,
        scratch_types=[
            pltpu.VMEM((_CH2, _B), jnp.int32),
            pltpu.VMEM((_B, _DW), jnp.float32),
            pltpu.VMEM_SHARED((_N, _DW), jnp.float32),
            pltpu.SemaphoreType.DMA,
        ],
    )
    def k(dst_hbm, ones_hbm, zeros_hbm, out_hbm, didx, ones_v, acc, ssem):
        c = lax.axis_index("c")
        s = lax.axis_index("s")
        pltpu.sync_copy(dst_hbm.at[s, pl.ds(c * _CH2, _CH2)], didx)
        pltpu.sync_copy(ones_hbm, ones_v)

        @pl.when(s < _NSTAGE)
        def _():
            pltpu.sync_copy(zeros_hbm.at[pl.ds(s * _RPT, _RPT)],
                            acc.at[pl.ds(s * _RPT, _RPT)])

        plsc.subcore_barrier()

        def body(g, carry):
            base = g * _K
            for b in range(_K):
                pltpu.async_copy(ones_v, acc.at[didx.at[base + b]], ssem,
                                 add=True)

            @pl.when(g > 0)
            def _():
                for b in range(_K):
                    pltpu.make_async_copy(
                        zeros_hbm.at[pl.ds(0, _B)], ones_v, ssem).wait()

            return carry

        lax.fori_loop(0, 2 * _GG2, body, 0)
        for b in range(_K):
            pltpu.make_async_copy(zeros_hbm.at[pl.ds(0, _B)], ones_v,
                                  ssem).wait()
        plsc.subcore_barrier()

        @pl.when(s < _NSTAGE)
        def _():
            pltpu.sync_copy(acc.at[pl.ds(s * _RPT, _RPT)],
                            out_hbm.at[c, pl.ds(s * _RPT, _RPT)])

    return k(dst3, ones_rows, zeros_d)


def _sc_gcn2(hs, dinv16, b1row, src3, dst3, zeros_h):
    mesh = plsc.VectorSubcoreMesh(core_axis_name="c", subcore_axis_name="s")

    @functools.partial(
        pl.kernel,
        out_type=[
            jax.ShapeDtypeStruct((_NC, _N, _H), jnp.float32),
            jax.ShapeDtypeStruct((_N, _H), jnp.float32),
        ],
        mesh=mesh,
        compiler_params=_SC_PARAMS,
        scratch_types=[
            pltpu.VMEM((_CHT, _B), jnp.int32),
            pltpu.VMEM((_CHT, _B), jnp.int32),
            pltpu.VMEM((2, _K, _B, _H), jnp.float32),
            pltpu.VMEM((_RPC, _H), jnp.float32),
            pltpu.VMEM((_RPC, _H), jnp.float32),
            pltpu.VMEM((_RPC, _H), jnp.float32),
            pltpu.VMEM((_RPC, _H), jnp.float32),
            pltpu.VMEM((1, _H), jnp.float32),
            pltpu.VMEM_SHARED((_N, _H), jnp.float32),
            pltpu.VMEM_SHARED((_N, _H), jnp.float32),
            pltpu.SemaphoreType.DMA,
            pltpu.SemaphoreType.DMA,
        ],
    )
    def k(hs_hbm, di_hbm, b1_hbm, src_hbm, dst_hbm, zeros_hbm,
          out_hbm, a_hbm, sidx, didx, rows,
          cs1, chs, cdi, ca, cb1, acc, tshared, gsem, ssem):
        c = lax.axis_index("c")
        s = lax.axis_index("s")
        pltpu.sync_copy(src_hbm.at[s], sidx)
        pltpu.sync_copy(dst_hbm.at[s], didx)

        @pl.when(s < _NSTAGE)
        def _():
            pltpu.sync_copy(zeros_hbm.at[pl.ds(s * _RPT, _RPT)],
                            acc.at[pl.ds(s * _RPT, _RPT)])
            pltpu.sync_copy(hs_hbm.at[pl.ds(s * _RPT, _RPT)],
                            tshared.at[pl.ds(s * _RPT, _RPT)])

        plsc.subcore_barrier()

        def fire_gathers(cb, buf):
            for b in range(_K):
                pltpu.async_copy(tshared.at[sidx.at[cb + b]],
                                 rows.at[buf, b], gsem)

        def fire_scatters(cb, buf):
            for b in range(_K):
                pltpu.async_copy(rows.at[buf, b], acc.at[didx.at[cb + b]],
                                 ssem, add=True)

        def drain(sem):
            for b in range(_K):
                pltpu.make_async_copy(zeros_hbm.at[pl.ds(0, _B)],
                                      rows.at[0, b], sem).wait()

        def run_pass(chunk0, npairs):
            fire_gathers(chunk0, 0)

            def body(gg, carry):
                g0 = chunk0 + 2 * gg * _K

                drain(gsem)
                fire_scatters(g0, 0)

                @pl.when(gg > 0)
                def _():
                    drain(ssem)

                fire_gathers(g0 + _K, 1)
                drain(gsem)
                fire_scatters(g0 + _K, 1)
                drain(ssem)

                @pl.when(gg + 1 < npairs)
                def _():
                    fire_gathers(g0 + 2 * _K, 0)

                return carry

            lax.fori_loop(0, npairs, body, 0)
            drain(ssem)

        run_pass(0, _GG1)
        plsc.subcore_barrier()

        base = s * _RPC
        pltpu.sync_copy(acc.at[pl.ds(base, _RPC)], cs1)
        pltpu.sync_copy(tshared.at[pl.ds(base, _RPC)], chs)
        pltpu.sync_copy(di_hbm.at[pl.ds(base, _RPC)], cdi)
        pltpu.sync_copy(b1_hbm, cb1)

        @plsc.parallel_loop(0, _RPC, unroll=4)
        def _(i):
            di = cdi[i]
            t = (cs1[i] + chs[i]) * di
            ca[i] = jnp.maximum(t + cb1[0], 0.0) * di

        pltpu.sync_copy(ca, tshared.at[pl.ds(base, _RPC)])
        pltpu.sync_copy(zeros_hbm.at[pl.ds(base, _RPC)],
                        acc.at[pl.ds(base, _RPC)])

        @pl.when(c == 0)
        def _():
            pltpu.sync_copy(ca, a_hbm.at[pl.ds(base, _RPC)])

        plsc.subcore_barrier()

        run_pass(c * _CH2, _GG2)
        plsc.subcore_barrier()

        @pl.when(s < _NSTAGE)
        def _():
            pltpu.sync_copy(acc.at[pl.ds(s * _RPT, _RPT)],
                            out_hbm.at[c, pl.ds(s * _RPT, _RPT)])

    return k(hs, dinv16, b1row, src3, dst3, zeros_h)


def _tc_pre(x, W1, degp):
    def body(x_ref, w_ref, deg_ref, hs_ref, di_ref):
        h = jnp.dot(x_ref[...], w_ref[...], preferred_element_type=jnp.float32)
        deg = deg_ref[0, :, 0:1] + deg_ref[1, :, 0:1] + 1.0
        dinv = lax.rsqrt(deg)
        hs_ref[...] = h * dinv
        di_ref[...] = jnp.broadcast_to(dinv, (_N, _H))

    return pl.pallas_call(
        body,
        out_shape=[
            jax.ShapeDtypeStruct((_N, _H), jnp.float32),
            jax.ShapeDtypeStruct((_N, _H), jnp.float32),
        ],
    )(x, W1, degp)


def _tc_out(S2, a, dinv16, W2, b2):
    def body(s_ref, a_ref, di_ref, w_ref, b_ref, o_ref):
        t = (s_ref[0] + s_ref[1] + a_ref[...]) * di_ref[...]
        z = jnp.dot(t, w_ref[...], preferred_element_type=jnp.float32) + b_ref[...]
        m = jnp.max(z, axis=1, keepdims=True)
        e = z - m
        lse = jnp.log(jnp.sum(jnp.exp(e), axis=1, keepdims=True))
        o_ref[...] = e - lse

    return pl.pallas_call(
        body,
        out_shape=jax.ShapeDtypeStruct((_N, _C), jnp.float32),
    )(S2, a, dinv16, W2, b2)


def kernel(x, edge_index, W1, b1, W2, b2):
    src3 = edge_index[0].reshape(_NS, _CHT, _B)
    dst3 = edge_index[1].reshape(_NS, _CHT, _B)
    ones_rows = jnp.ones((_B, _DW), jnp.float32)
    zeros_d = jnp.zeros((_N, _DW), jnp.float32)
    zeros_h = jnp.zeros((_N, _H), jnp.float32)

    degp = _sc_degree(dst3, ones_rows, zeros_d)
    hs, dinv16 = _tc_pre(x, W1, degp)
    S2, a = _sc_gcn2(hs, dinv16, b1.reshape(1, _H), src3, dst3, zeros_h)
    return _tc_out(S2, a, dinv16, W2, b2.reshape(1, _C))


# speedup vs baseline: 1.0649x; 1.0649x over previous
import functools

import jax
import jax.numpy as jnp
from jax import lax
from jax.experimental import pallas as pl
from jax.experimental.pallas import tpu as pltpu
from jax.experimental.pallas import tpu_sc as plsc

_N, _E, _FIN, _H, _C = 10000, 320000, 128, 16, 40
_NC, _NS = 2, 16
_NW = _NC * _NS
_EPT = _E // _NW
_B = 100
_CH = _EPT // _B
_K = 5
_G = _CH // _K
_GG = _G // 2
_RPT = 1000
_NSTAGE = _N // _RPT
_DW = 16

_SC_PARAMS = pltpu.CompilerParams(use_tc_tiling_on_sc=False)


def _sc_degree(dst3, ones_rows, zeros_d):
    mesh = plsc.VectorSubcoreMesh(core_axis_name="c", subcore_axis_name="s")

    @functools.partial(
        pl.kernel,
        out_type=jax.ShapeDtypeStruct((_NC, _N, _DW), jnp.float32),
        mesh=mesh,
        compiler_params=_SC_PARAMS,
        scratch_types=[
            pltpu.VMEM((_CH, _B), jnp.int32),
            pltpu.VMEM((_B, _DW), jnp.float32),
            pltpu.VMEM_SHARED((_N, _DW), jnp.float32),
            pltpu.SemaphoreType.DMA,
        ],
    )
    def k(dst_hbm, ones_hbm, zeros_hbm, out_hbm, didx, ones_v, acc, ssem):
        c = lax.axis_index("c")
        s = lax.axis_index("s")
        w = s * _NC + c
        pltpu.sync_copy(dst_hbm.at[w], didx)
        pltpu.sync_copy(ones_hbm, ones_v)

        @pl.when(s < _NSTAGE)
        def _():
            pltpu.sync_copy(zeros_hbm.at[pl.ds(s * _RPT, _RPT)],
                            acc.at[pl.ds(s * _RPT, _RPT)])

        plsc.subcore_barrier()

        def body(g, carry):
            base = g * _K
            for b in range(_K):
                pltpu.async_copy(ones_v, acc.at[didx.at[base + b]], ssem,
                                 add=True)

            @pl.when(g > 0)
            def _():
                for b in range(_K):
                    pltpu.make_async_copy(
                        zeros_hbm.at[pl.ds(0, _B)], ones_v, ssem).wait()

            return carry

        lax.fori_loop(0, _G, body, 0)
        for b in range(_K):
            pltpu.make_async_copy(zeros_hbm.at[pl.ds(0, _B)], ones_v,
                                  ssem).wait()
        plsc.subcore_barrier()

        @pl.when(s < _NSTAGE)
        def _():
            pltpu.sync_copy(acc.at[pl.ds(s * _RPT, _RPT)],
                            out_hbm.at[c, pl.ds(s * _RPT, _RPT)])

    return k(dst3, ones_rows, zeros_d)


def _sc_scatter16(table, src3, dst3, zeros_h):
    mesh = plsc.VectorSubcoreMesh(core_axis_name="c", subcore_axis_name="s")

    @functools.partial(
        pl.kernel,
        out_type=jax.ShapeDtypeStruct((_NC, _N, _H), jnp.float32),
        mesh=mesh,
        compiler_params=_SC_PARAMS,
        scratch_types=[
            pltpu.VMEM((_CH, _B), jnp.int32),
            pltpu.VMEM((_CH, _B), jnp.int32),
            pltpu.VMEM((2, _K, _B, _H), jnp.float32),
            pltpu.VMEM_SHARED((_N, _H), jnp.float32),
            pltpu.VMEM_SHARED((_N, _H), jnp.float32),
            pltpu.SemaphoreType.DMA,
            pltpu.SemaphoreType.DMA,
        ],
    )
    def k(tab_hbm, src_hbm, dst_hbm, zeros_hbm, out_hbm,
          sidx, didx, rows, acc, tshared, gsem, ssem):
        c = lax.axis_index("c")
        s = lax.axis_index("s")
        w = s * _NC + c
        pltpu.sync_copy(src_hbm.at[w], sidx)
        pltpu.sync_copy(dst_hbm.at[w], didx)

        @pl.when(s < _NSTAGE)
        def _():
            pltpu.sync_copy(zeros_hbm.at[pl.ds(s * _RPT, _RPT)],
                            acc.at[pl.ds(s * _RPT, _RPT)])
            pltpu.sync_copy(tab_hbm.at[pl.ds(s * _RPT, _RPT)],
                            tshared.at[pl.ds(s * _RPT, _RPT)])

        plsc.subcore_barrier()

        def fire_gathers(base, buf):
            for b in range(_K):
                pltpu.async_copy(tshared.at[sidx.at[base + b]],
                                 rows.at[buf, b], gsem)

        def fire_scatters(base, buf):
            for b in range(_K):
                pltpu.async_copy(rows.at[buf, b], acc.at[didx.at[base + b]],
                                 ssem, add=True)

        def drain(sem):
            for b in range(_K):
                pltpu.make_async_copy(zeros_hbm.at[pl.ds(0, _B)],
                                      rows.at[0, b], sem).wait()

        fire_gathers(0, 0)

        def body(gg, carry):
            g0 = 2 * gg * _K

            drain(gsem)
            fire_scatters(g0, 0)

            @pl.when(gg > 0)
            def _():
                drain(ssem)

            fire_gathers(g0 + _K, 1)
            drain(gsem)
            fire_scatters(g0 + _K, 1)
            drain(ssem)

            @pl.when(gg + 1 < _GG)
            def _():
                fire_gathers(g0 + 2 * _K, 0)

            return carry

        lax.fori_loop(0, _GG, body, 0)
        drain(ssem)
        plsc.subcore_barrier()

        @pl.when(s < _NSTAGE)
        def _():
            pltpu.sync_copy(acc.at[pl.ds(s * _RPT, _RPT)],
                            out_hbm.at[c, pl.ds(s * _RPT, _RPT)])

    return k(table, src3, dst3, zeros_h)


_RPC = _N // _NS


def _sc_scatter16_mid(S1p, hs, dinv16, b1row, src3, dst3, zeros_h):
    mesh = plsc.VectorSubcoreMesh(core_axis_name="c", subcore_axis_name="s")

    @functools.partial(
        pl.kernel,
        out_type=[
            jax.ShapeDtypeStruct((_NC, _N, _H), jnp.float32),
            jax.ShapeDtypeStruct((_N, _H), jnp.float32),
        ],
        mesh=mesh,
        compiler_params=_SC_PARAMS,
        scratch_types=[
            pltpu.VMEM((_CH, _B), jnp.int32),
            pltpu.VMEM((_CH, _B), jnp.int32),
            pltpu.VMEM((2, _K, _B, _H), jnp.float32),
            pltpu.VMEM((_RPC, _H), jnp.float32),
            pltpu.VMEM((_RPC, _H), jnp.float32),
            pltpu.VMEM((_RPC, _H), jnp.float32),
            pltpu.VMEM((_RPC, _H), jnp.float32),
            pltpu.VMEM((_RPC, _H), jnp.float32),
            pltpu.VMEM((1, _H), jnp.float32),
            pltpu.VMEM_SHARED((_N, _H), jnp.float32),
            pltpu.VMEM_SHARED((_N, _H), jnp.float32),
            pltpu.SemaphoreType.DMA,
            pltpu.SemaphoreType.DMA,
        ],
    )
    def k(s1_hbm, hs_hbm, di_hbm, b1_hbm, src_hbm, dst_hbm, zeros_hbm,
          out_hbm, a_hbm, sidx, didx, rows,
          cs1a, cs1b, chs, cdi, ca, cb1, acc, tshared, gsem, ssem):
        c = lax.axis_index("c")
        s = lax.axis_index("s")
        w = s * _NC + c
        pltpu.sync_copy(src_hbm.at[w], sidx)
        pltpu.sync_copy(dst_hbm.at[w], didx)

        @pl.when(s < _NSTAGE)
        def _():
            pltpu.sync_copy(zeros_hbm.at[pl.ds(s * _RPT, _RPT)],
                            acc.at[pl.ds(s * _RPT, _RPT)])

        base = s * _RPC
        pltpu.sync_copy(s1_hbm.at[0, pl.ds(base, _RPC)], cs1a)
        pltpu.sync_copy(s1_hbm.at[1, pl.ds(base, _RPC)], cs1b)
        pltpu.sync_copy(hs_hbm.at[pl.ds(base, _RPC)], chs)
        pltpu.sync_copy(di_hbm.at[pl.ds(base, _RPC)], cdi)
        pltpu.sync_copy(b1_hbm, cb1)

        @plsc.parallel_loop(0, _RPC, unroll=4)
        def _(i):
            di = cdi[i]
            t = (cs1a[i] + cs1b[i] + chs[i]) * di
            ca[i] = jnp.maximum(t + cb1[0], 0.0) * di
        pltpu.sync_copy(ca, tshared.at[pl.ds(base, _RPC)])

        @pl.when(c == 0)
        def _():
            pltpu.sync_copy(ca, a_hbm.at[pl.ds(base, _RPC)])

        plsc.subcore_barrier()

        def fire_gathers(gbase, buf):
            for b in range(_K):
                pltpu.async_copy(tshared.at[sidx.at[gbase + b]],
                                 rows.at[buf, b], gsem)

        def fire_scatters(gbase, buf):
            for b in range(_K):
                pltpu.async_copy(rows.at[buf, b], acc.at[didx.at[gbase + b]],
                                 ssem, add=True)

        def drain(sem):
            for b in range(_K):
                pltpu.make_async_copy(zeros_hbm.at[pl.ds(0, _B)],
                                      rows.at[0, b], sem).wait()

        fire_gathers(0, 0)

        def body(gg, carry):
            g0 = 2 * gg * _K

            drain(gsem)
            fire_scatters(g0, 0)

            @pl.when(gg > 0)
            def _():
                drain(ssem)

            fire_gathers(g0 + _K, 1)
            drain(gsem)
            fire_scatters(g0 + _K, 1)
            drain(ssem)

            @pl.when(gg + 1 < _GG)
            def _():
                fire_gathers(g0 + 2 * _K, 0)

            return carry

        lax.fori_loop(0, _GG, body, 0)
        drain(ssem)
        plsc.subcore_barrier()

        @pl.when(s < _NSTAGE)
        def _():
            pltpu.sync_copy(acc.at[pl.ds(s * _RPT, _RPT)],
                            out_hbm.at[c, pl.ds(s * _RPT, _RPT)])

    return k(S1p, hs, dinv16, b1row, src3, dst3, zeros_h)


def _tc_pre(x, W1, degp):
    def body(x_ref, w_ref, deg_ref, hs_ref, di_ref):
        h = jnp.dot(x_ref[...], w_ref[...], preferred_element_type=jnp.float32)
        deg = deg_ref[0, :, 0:1] + deg_ref[1, :, 0:1] + 1.0
        dinv = lax.rsqrt(deg)
        hs_ref[...] = h * dinv
        di_ref[...] = jnp.broadcast_to(dinv, (_N, _H))

    return pl.pallas_call(
        body,
        out_shape=[
            jax.ShapeDtypeStruct((_N, _H), jnp.float32),
            jax.ShapeDtypeStruct((_N, _H), jnp.float32),
        ],
    )(x, W1, degp)


def _tc_mid(S1, hs, dinv16, b1):
    def body(s_ref, hs_ref, di_ref, b_ref, o_ref):
        t = (s_ref[0] + s_ref[1] + hs_ref[...]) * di_ref[...]
        o_ref[...] = jnp.maximum(t + b_ref[...], 0.0) * di_ref[...]

    return pl.pallas_call(
        body,
        out_shape=jax.ShapeDtypeStruct((_N, _H), jnp.float32),
    )(S1, hs, dinv16, b1)


def _tc_out(S2, a, dinv16, W2, b2):
    def body(s_ref, a_ref, di_ref, w_ref, b_ref, o_ref):
        t = (s_ref[0] + s_ref[1] + a_ref[...]) * di_ref[...]
        z = jnp.dot(t, w_ref[...], preferred_element_type=jnp.float32) + b_ref[...]
        m = jnp.max(z, axis=1, keepdims=True)
        e = z - m
        lse = jnp.log(jnp.sum(jnp.exp(e), axis=1, keepdims=True))
        o_ref[...] = e - lse

    return pl.pallas_call(
        body,
        out_shape=jax.ShapeDtypeStruct((_N, _C), jnp.float32),
    )(S2, a, dinv16, W2, b2)


def kernel(x, edge_index, W1, b1, W2, b2):
    src3 = edge_index[0].reshape(_NW, _CH, _B)
    dst3 = edge_index[1].reshape(_NW, _CH, _B)
    ones_rows = jnp.ones((_B, _DW), jnp.float32)
    zeros_d = jnp.zeros((_N, _DW), jnp.float32)
    zeros_h = jnp.zeros((_N, _H), jnp.float32)

    degp = _sc_degree(dst3, ones_rows, zeros_d)
    hs, dinv16 = _tc_pre(x, W1, degp)
    S1 = _sc_scatter16(hs, src3, dst3, zeros_h)
    S2, a = _sc_scatter16_mid(S1, hs, dinv16, b1.reshape(1, _H),
                              src3, dst3, zeros_h)
    return _tc_out(S2, a, dinv16, W2, b2.reshape(1, _C))


# speedup vs baseline: 1.0759x; 1.0104x over previous
import functools

import jax
import jax.numpy as jnp
from jax import lax
from jax.experimental import pallas as pl
from jax.experimental.pallas import tpu as pltpu
from jax.experimental.pallas import tpu_sc as plsc

_N, _E, _FIN, _H, _C = 10000, 320000, 128, 16, 40
_NC, _NS = 2, 16
_NW = _NC * _NS
_EPT = _E // _NW
_B = 125
_CH = _EPT // _B
_K = 5
_G = _CH // _K
_GG = _G // 2
_RPT = 1000
_NSTAGE = _N // _RPT
_DW = 16

_SC_PARAMS = pltpu.CompilerParams(use_tc_tiling_on_sc=False)


def _sc_degree(dst3, ones_rows, zeros_d):
    mesh = plsc.VectorSubcoreMesh(core_axis_name="c", subcore_axis_name="s")

    @functools.partial(
        pl.kernel,
        out_type=jax.ShapeDtypeStruct((_NC, _N, _DW), jnp.float32),
        mesh=mesh,
        compiler_params=_SC_PARAMS,
        scratch_types=[
            pltpu.VMEM((_CH, _B), jnp.int32),
            pltpu.VMEM((_B, _DW), jnp.float32),
            pltpu.VMEM_SHARED((_N, _DW), jnp.float32),
            pltpu.SemaphoreType.DMA,
        ],
    )
    def k(dst_hbm, ones_hbm, zeros_hbm, out_hbm, didx, ones_v, acc, ssem):
        c = lax.axis_index("c")
        s = lax.axis_index("s")
        w = s * _NC + c
        pltpu.sync_copy(dst_hbm.at[w], didx)
        pltpu.sync_copy(ones_hbm, ones_v)

        @pl.when(s < _NSTAGE)
        def _():
            pltpu.sync_copy(zeros_hbm.at[pl.ds(s * _RPT, _RPT)],
                            acc.at[pl.ds(s * _RPT, _RPT)])

        plsc.subcore_barrier()

        def body(g, carry):
            base = g * _K
            for b in range(_K):
                pltpu.async_copy(ones_v, acc.at[didx.at[base + b]], ssem,
                                 add=True)

            @pl.when(g > 0)
            def _():
                for b in range(_K):
                    pltpu.make_async_copy(
                        zeros_hbm.at[pl.ds(0, _B)], ones_v, ssem).wait()

            return carry

        lax.fori_loop(0, _G, body, 0)
        for b in range(_K):
            pltpu.make_async_copy(zeros_hbm.at[pl.ds(0, _B)], ones_v,
                                  ssem).wait()
        plsc.subcore_barrier()

        @pl.when(s < _NSTAGE)
        def _():
            pltpu.sync_copy(acc.at[pl.ds(s * _RPT, _RPT)],
                            out_hbm.at[c, pl.ds(s * _RPT, _RPT)])

    return k(dst3, ones_rows, zeros_d)


def _sc_scatter16(table, src3, dst3, zeros_h):
    mesh = plsc.VectorSubcoreMesh(core_axis_name="c", subcore_axis_name="s")

    @functools.partial(
        pl.kernel,
        out_type=jax.ShapeDtypeStruct((_NC, _N, _H), jnp.float32),
        mesh=mesh,
        compiler_params=_SC_PARAMS,
        scratch_types=[
            pltpu.VMEM((_CH, _B), jnp.int32),
            pltpu.VMEM((_CH, _B), jnp.int32),
            pltpu.VMEM((2, _K, _B, _H), jnp.float32),
            pltpu.VMEM_SHARED((_N, _H), jnp.float32),
            pltpu.VMEM_SHARED((_N, _H), jnp.float32),
            pltpu.SemaphoreType.DMA,
            pltpu.SemaphoreType.DMA,
        ],
    )
    def k(tab_hbm, src_hbm, dst_hbm, zeros_hbm, out_hbm,
          sidx, didx, rows, acc, tshared, gsem, ssem):
        c = lax.axis_index("c")
        s = lax.axis_index("s")
        w = s * _NC + c
        pltpu.sync_copy(src_hbm.at[w], sidx)
        pltpu.sync_copy(dst_hbm.at[w], didx)

        @pl.when(s < _NSTAGE)
        def _():
            pltpu.sync_copy(zeros_hbm.at[pl.ds(s * _RPT, _RPT)],
                            acc.at[pl.ds(s * _RPT, _RPT)])
            pltpu.sync_copy(tab_hbm.at[pl.ds(s * _RPT, _RPT)],
                            tshared.at[pl.ds(s * _RPT, _RPT)])

        plsc.subcore_barrier()

        def fire_gathers(base, buf):
            for b in range(_K):
                pltpu.async_copy(tshared.at[sidx.at[base + b]],
                                 rows.at[buf, b], gsem)

        def fire_scatters(base, buf):
            for b in range(_K):
                pltpu.async_copy(rows.at[buf, b], acc.at[didx.at[base + b]],
                                 ssem, add=True)

        def drain(sem):
            for b in range(_K):
                pltpu.make_async_copy(zeros_hbm.at[pl.ds(0, _B)],
                                      rows.at[0, b], sem).wait()

        fire_gathers(0, 0)

        def body(gg, carry):
            g0 = 2 * gg * _K

            drain(gsem)
            fire_scatters(g0, 0)

            @pl.when(gg > 0)
            def _():
                drain(ssem)

            fire_gathers(g0 + _K, 1)
            drain(gsem)
            fire_scatters(g0 + _K, 1)
            drain(ssem)

            @pl.when(gg + 1 < _GG)
            def _():
                fire_gathers(g0 + 2 * _K, 0)

            return carry

        lax.fori_loop(0, _GG, body, 0)
        drain(ssem)
        plsc.subcore_barrier()

        @pl.when(s < _NSTAGE)
        def _():
            pltpu.sync_copy(acc.at[pl.ds(s * _RPT, _RPT)],
                            out_hbm.at[c, pl.ds(s * _RPT, _RPT)])

    return k(table, src3, dst3, zeros_h)


_RPC = _N // _NS


def _sc_scatter16_mid(S1p, hs, dinv16, b1row, src3, dst3, zeros_h):
    mesh = plsc.VectorSubcoreMesh(core_axis_name="c", subcore_axis_name="s")

    @functools.partial(
        pl.kernel,
        out_type=[
            jax.ShapeDtypeStruct((_NC, _N, _H), jnp.float32),
            jax.ShapeDtypeStruct((_N, _H), jnp.float32),
        ],
        mesh=mesh,
        compiler_params=_SC_PARAMS,
        scratch_types=[
            pltpu.VMEM((_CH, _B), jnp.int32),
            pltpu.VMEM((_CH, _B), jnp.int32),
            pltpu.VMEM((2, _K, _B, _H), jnp.float32),
            pltpu.VMEM((_RPC, _H), jnp.float32),
            pltpu.VMEM((_RPC, _H), jnp.float32),
            pltpu.VMEM((_RPC, _H), jnp.float32),
            pltpu.VMEM((_RPC, _H), jnp.float32),
            pltpu.VMEM((_RPC, _H), jnp.float32),
            pltpu.VMEM((1, _H), jnp.float32),
            pltpu.VMEM_SHARED((_N, _H), jnp.float32),
            pltpu.VMEM_SHARED((_N, _H), jnp.float32),
            pltpu.SemaphoreType.DMA,
            pltpu.SemaphoreType.DMA,
        ],
    )
    def k(s1_hbm, hs_hbm, di_hbm, b1_hbm, src_hbm, dst_hbm, zeros_hbm,
          out_hbm, a_hbm, sidx, didx, rows,
          cs1a, cs1b, chs, cdi, ca, cb1, acc, tshared, gsem, ssem):
        c = lax.axis_index("c")
        s = lax.axis_index("s")
        w = s * _NC + c
        pltpu.sync_copy(src_hbm.at[w], sidx)
        pltpu.sync_copy(dst_hbm.at[w], didx)

        @pl.when(s < _NSTAGE)
        def _():
            pltpu.sync_copy(zeros_hbm.at[pl.ds(s * _RPT, _RPT)],
                            acc.at[pl.ds(s * _RPT, _RPT)])

        base = s * _RPC
        pltpu.sync_copy(s1_hbm.at[0, pl.ds(base, _RPC)], cs1a)
        pltpu.sync_copy(s1_hbm.at[1, pl.ds(base, _RPC)], cs1b)
        pltpu.sync_copy(hs_hbm.at[pl.ds(base, _RPC)], chs)
        pltpu.sync_copy(di_hbm.at[pl.ds(base, _RPC)], cdi)
        pltpu.sync_copy(b1_hbm, cb1)

        @plsc.parallel_loop(0, _RPC, unroll=4)
        def _(i):
            di = cdi[i]
            t = (cs1a[i] + cs1b[i] + chs[i]) * di
            ca[i] = jnp.maximum(t + cb1[0], 0.0) * di
        pltpu.sync_copy(ca, tshared.at[pl.ds(base, _RPC)])

        @pl.when(c == 0)
        def _():
            pltpu.sync_copy(ca, a_hbm.at[pl.ds(base, _RPC)])

        plsc.subcore_barrier()

        def fire_gathers(gbase, buf):
            for b in range(_K):
                pltpu.async_copy(tshared.at[sidx.at[gbase + b]],
                                 rows.at[buf, b], gsem)

        def fire_scatters(gbase, buf):
            for b in range(_K):
                pltpu.async_copy(rows.at[buf, b], acc.at[didx.at[gbase + b]],
                                 ssem, add=True)

        def drain(sem):
            for b in range(_K):
                pltpu.make_async_copy(zeros_hbm.at[pl.ds(0, _B)],
                                      rows.at[0, b], sem).wait()

        fire_gathers(0, 0)

        def body(gg, carry):
            g0 = 2 * gg * _K

            drain(gsem)
            fire_scatters(g0, 0)

            @pl.when(gg > 0)
            def _():
                drain(ssem)

            fire_gathers(g0 + _K, 1)
            drain(gsem)
            fire_scatters(g0 + _K, 1)
            drain(ssem)

            @pl.when(gg + 1 < _GG)
            def _():
                fire_gathers(g0 + 2 * _K, 0)

            return carry

        lax.fori_loop(0, _GG, body, 0)
        drain(ssem)
        plsc.subcore_barrier()

        @pl.when(s < _NSTAGE)
        def _():
            pltpu.sync_copy(acc.at[pl.ds(s * _RPT, _RPT)],
                            out_hbm.at[c, pl.ds(s * _RPT, _RPT)])

    return k(S1p, hs, dinv16, b1row, src3, dst3, zeros_h)


def _tc_pre(x, W1, degp):
    def body(x_ref, w_ref, deg_ref, hs_ref, di_ref):
        h = jnp.dot(x_ref[...], w_ref[...], preferred_element_type=jnp.float32)
        deg = deg_ref[0, :, 0:1] + deg_ref[1, :, 0:1] + 1.0
        dinv = lax.rsqrt(deg)
        hs_ref[...] = h * dinv
        di_ref[...] = jnp.broadcast_to(dinv, (_N, _H))

    return pl.pallas_call(
        body,
        out_shape=[
            jax.ShapeDtypeStruct((_N, _H), jnp.float32),
            jax.ShapeDtypeStruct((_N, _H), jnp.float32),
        ],
    )(x, W1, degp)


def _tc_mid(S1, hs, dinv16, b1):
    def body(s_ref, hs_ref, di_ref, b_ref, o_ref):
        t = (s_ref[0] + s_ref[1] + hs_ref[...]) * di_ref[...]
        o_ref[...] = jnp.maximum(t + b_ref[...], 0.0) * di_ref[...]

    return pl.pallas_call(
        body,
        out_shape=jax.ShapeDtypeStruct((_N, _H), jnp.float32),
    )(S1, hs, dinv16, b1)


def _tc_out(S2, a, dinv16, W2, b2):
    def body(s_ref, a_ref, di_ref, w_ref, b_ref, o_ref):
        t = (s_ref[0] + s_ref[1] + a_ref[...]) * di_ref[...]
        z = jnp.dot(t, w_ref[...], preferred_element_type=jnp.float32) + b_ref[...]
        m = jnp.max(z, axis=1, keepdims=True)
        e = z - m
        lse = jnp.log(jnp.sum(jnp.exp(e), axis=1, keepdims=True))
        o_ref[...] = e - lse

    return pl.pallas_call(
        body,
        out_shape=jax.ShapeDtypeStruct((_N, _C), jnp.float32),
    )(S2, a, dinv16, W2, b2)


def kernel(x, edge_index, W1, b1, W2, b2):
    src3 = edge_index[0].reshape(_NW, _CH, _B)
    dst3 = edge_index[1].reshape(_NW, _CH, _B)
    ones_rows = jnp.ones((_B, _DW), jnp.float32)
    zeros_d = jnp.zeros((_N, _DW), jnp.float32)
    zeros_h = jnp.zeros((_N, _H), jnp.float32)

    degp = _sc_degree(dst3, ones_rows, zeros_d)
    hs, dinv16 = _tc_pre(x, W1, degp)
    S1 = _sc_scatter16(hs, src3, dst3, zeros_h)
    S2, a = _sc_scatter16_mid(S1, hs, dinv16, b1.reshape(1, _H),
                              src3, dst3, zeros_h)
    return _tc_out(S2, a, dinv16, W2, b2.reshape(1, _C))


# speedup vs baseline: 1.0765x; 1.0005x over previous
import functools

import jax
import jax.numpy as jnp
from jax import lax
from jax.experimental import pallas as pl
from jax.experimental.pallas import tpu as pltpu
from jax.experimental.pallas import tpu_sc as plsc

_N, _E, _FIN, _H, _C = 10000, 320000, 128, 16, 40
_NC, _NS = 2, 16
_NW = _NC * _NS
_EPT = _E // _NW
_B = 100
_CH = _EPT // _B
_K = 10
_G = _CH // _K
_GG = _G // 2
_RPT = 1000
_NSTAGE = _N // _RPT
_DW = 16

_SC_PARAMS = pltpu.CompilerParams(use_tc_tiling_on_sc=False)


def _sc_degree(dst3, ones_rows, zeros_d):
    mesh = plsc.VectorSubcoreMesh(core_axis_name="c", subcore_axis_name="s")

    @functools.partial(
        pl.kernel,
        out_type=jax.ShapeDtypeStruct((_NC, _N, _DW), jnp.float32),
        mesh=mesh,
        compiler_params=_SC_PARAMS,
        scratch_types=[
            pltpu.VMEM((_CH, _B), jnp.int32),
            pltpu.VMEM((_B, _DW), jnp.float32),
            pltpu.VMEM_SHARED((_N, _DW), jnp.float32),
            pltpu.SemaphoreType.DMA,
        ],
    )
    def k(dst_hbm, ones_hbm, zeros_hbm, out_hbm, didx, ones_v, acc, ssem):
        c = lax.axis_index("c")
        s = lax.axis_index("s")
        w = s * _NC + c
        pltpu.sync_copy(dst_hbm.at[w], didx)
        pltpu.sync_copy(ones_hbm, ones_v)

        @pl.when(s < _NSTAGE)
        def _():
            pltpu.sync_copy(zeros_hbm.at[pl.ds(s * _RPT, _RPT)],
                            acc.at[pl.ds(s * _RPT, _RPT)])

        plsc.subcore_barrier()

        def body(g, carry):
            base = g * _K
            for b in range(_K):
                pltpu.async_copy(ones_v, acc.at[didx.at[base + b]], ssem,
                                 add=True)

            @pl.when(g > 0)
            def _():
                for b in range(_K):
                    pltpu.make_async_copy(
                        zeros_hbm.at[pl.ds(0, _B)], ones_v, ssem).wait()

            return carry

        lax.fori_loop(0, _G, body, 0)
        for b in range(_K):
            pltpu.make_async_copy(zeros_hbm.at[pl.ds(0, _B)], ones_v,
                                  ssem).wait()
        plsc.subcore_barrier()

        @pl.when(s < _NSTAGE)
        def _():
            pltpu.sync_copy(acc.at[pl.ds(s * _RPT, _RPT)],
                            out_hbm.at[c, pl.ds(s * _RPT, _RPT)])

    return k(dst3, ones_rows, zeros_d)


def _sc_scatter16(table, src3, dst3, zeros_h):
    mesh = plsc.VectorSubcoreMesh(core_axis_name="c", subcore_axis_name="s")

    @functools.partial(
        pl.kernel,
        out_type=jax.ShapeDtypeStruct((_NC, _N, _H), jnp.float32),
        mesh=mesh,
        compiler_params=_SC_PARAMS,
        scratch_types=[
            pltpu.VMEM((_CH, _B), jnp.int32),
            pltpu.VMEM((_CH, _B), jnp.int32),
            pltpu.VMEM((2, _K, _B, _H), jnp.float32),
            pltpu.VMEM_SHARED((_N, _H), jnp.float32),
            pltpu.VMEM_SHARED((_N, _H), jnp.float32),
            pltpu.SemaphoreType.DMA,
            pltpu.SemaphoreType.DMA,
        ],
    )
    def k(tab_hbm, src_hbm, dst_hbm, zeros_hbm, out_hbm,
          sidx, didx, rows, acc, tshared, gsem, ssem):
        c = lax.axis_index("c")
        s = lax.axis_index("s")
        w = s * _NC + c
        pltpu.sync_copy(src_hbm.at[w], sidx)
        pltpu.sync_copy(dst_hbm.at[w], didx)

        @pl.when(s < _NSTAGE)
        def _():
            pltpu.sync_copy(zeros_hbm.at[pl.ds(s * _RPT, _RPT)],
                            acc.at[pl.ds(s * _RPT, _RPT)])
            pltpu.sync_copy(tab_hbm.at[pl.ds(s * _RPT, _RPT)],
                            tshared.at[pl.ds(s * _RPT, _RPT)])

        plsc.subcore_barrier()

        def fire_gathers(base, buf):
            for b in range(_K):
                pltpu.async_copy(tshared.at[sidx.at[base + b]],
                                 rows.at[buf, b], gsem)

        def fire_scatters(base, buf):
            for b in range(_K):
                pltpu.async_copy(rows.at[buf, b], acc.at[didx.at[base + b]],
                                 ssem, add=True)

        def drain(sem):
            for b in range(_K):
                pltpu.make_async_copy(zeros_hbm.at[pl.ds(0, _B)],
                                      rows.at[0, b], sem).wait()

        fire_gathers(0, 0)

        def body(gg, carry):
            g0 = 2 * gg * _K

            drain(gsem)
            fire_scatters(g0, 0)

            @pl.when(gg > 0)
            def _():
                drain(ssem)

            fire_gathers(g0 + _K, 1)
            drain(gsem)
            fire_scatters(g0 + _K, 1)
            drain(ssem)

            @pl.when(gg + 1 < _GG)
            def _():
                fire_gathers(g0 + 2 * _K, 0)

            return carry

        lax.fori_loop(0, _GG, body, 0)
        drain(ssem)
        plsc.subcore_barrier()

        @pl.when(s < _NSTAGE)
        def _():
            pltpu.sync_copy(acc.at[pl.ds(s * _RPT, _RPT)],
                            out_hbm.at[c, pl.ds(s * _RPT, _RPT)])

    return k(table, src3, dst3, zeros_h)


_RPC = _N // _NS


def _sc_scatter16_mid(S1p, hs, dinv16, b1row, src3, dst3, zeros_h):
    mesh = plsc.VectorSubcoreMesh(core_axis_name="c", subcore_axis_name="s")

    @functools.partial(
        pl.kernel,
        out_type=[
            jax.ShapeDtypeStruct((_NC, _N, _H), jnp.float32),
            jax.ShapeDtypeStruct((_N, _H), jnp.float32),
        ],
        mesh=mesh,
        compiler_params=_SC_PARAMS,
        scratch_types=[
            pltpu.VMEM((_CH, _B), jnp.int32),
            pltpu.VMEM((_CH, _B), jnp.int32),
            pltpu.VMEM((2, _K, _B, _H), jnp.float32),
            pltpu.VMEM((_RPC, _H), jnp.float32),
            pltpu.VMEM((_RPC, _H), jnp.float32),
            pltpu.VMEM((_RPC, _H), jnp.float32),
            pltpu.VMEM((_RPC, _H), jnp.float32),
            pltpu.VMEM((_RPC, _H), jnp.float32),
            pltpu.VMEM((1, _H), jnp.float32),
            pltpu.VMEM_SHARED((_N, _H), jnp.float32),
            pltpu.VMEM_SHARED((_N, _H), jnp.float32),
            pltpu.SemaphoreType.DMA,
            pltpu.SemaphoreType.DMA,
        ],
    )
    def k(s1_hbm, hs_hbm, di_hbm, b1_hbm, src_hbm, dst_hbm, zeros_hbm,
          out_hbm, a_hbm, sidx, didx, rows,
          cs1a, cs1b, chs, cdi, ca, cb1, acc, tshared, gsem, ssem):
        c = lax.axis_index("c")
        s = lax.axis_index("s")
        w = s * _NC + c
        pltpu.sync_copy(src_hbm.at[w], sidx)
        pltpu.sync_copy(dst_hbm.at[w], didx)

        @pl.when(s < _NSTAGE)
        def _():
            pltpu.sync_copy(zeros_hbm.at[pl.ds(s * _RPT, _RPT)],
                            acc.at[pl.ds(s * _RPT, _RPT)])

        base = s * _RPC
        pltpu.sync_copy(s1_hbm.at[0, pl.ds(base, _RPC)], cs1a)
        pltpu.sync_copy(s1_hbm.at[1, pl.ds(base, _RPC)], cs1b)
        pltpu.sync_copy(hs_hbm.at[pl.ds(base, _RPC)], chs)
        pltpu.sync_copy(di_hbm.at[pl.ds(base, _RPC)], cdi)
        pltpu.sync_copy(b1_hbm, cb1)

        @plsc.parallel_loop(0, _RPC, unroll=4)
        def _(i):
            di = cdi[i]
            t = (cs1a[i] + cs1b[i] + chs[i]) * di
            ca[i] = jnp.maximum(t + cb1[0], 0.0) * di
        pltpu.sync_copy(ca, tshared.at[pl.ds(base, _RPC)])

        @pl.when(c == 0)
        def _():
            pltpu.sync_copy(ca, a_hbm.at[pl.ds(base, _RPC)])

        plsc.subcore_barrier()

        def fire_gathers(gbase, buf):
            for b in range(_K):
                pltpu.async_copy(tshared.at[sidx.at[gbase + b]],
                                 rows.at[buf, b], gsem)

        def fire_scatters(gbase, buf):
            for b in range(_K):
                pltpu.async_copy(rows.at[buf, b], acc.at[didx.at[gbase + b]],
                                 ssem, add=True)

        def drain(sem):
            for b in range(_K):
                pltpu.make_async_copy(zeros_hbm.at[pl.ds(0, _B)],
                                      rows.at[0, b], sem).wait()

        fire_gathers(0, 0)

        def body(gg, carry):
            g0 = 2 * gg * _K

            drain(gsem)
            fire_scatters(g0, 0)

            @pl.when(gg > 0)
            def _():
                drain(ssem)

            fire_gathers(g0 + _K, 1)
            drain(gsem)
            fire_scatters(g0 + _K, 1)
            drain(ssem)

            @pl.when(gg + 1 < _GG)
            def _():
                fire_gathers(g0 + 2 * _K, 0)

            return carry

        lax.fori_loop(0, _GG, body, 0)
        drain(ssem)
        plsc.subcore_barrier()

        @pl.when(s < _NSTAGE)
        def _():
            pltpu.sync_copy(acc.at[pl.ds(s * _RPT, _RPT)],
                            out_hbm.at[c, pl.ds(s * _RPT, _RPT)])

    return k(S1p, hs, dinv16, b1row, src3, dst3, zeros_h)


def _tc_pre(x, W1, degp):
    def body(x_ref, w_ref, deg_ref, hs_ref, di_ref):
        h = jnp.dot(x_ref[...], w_ref[...], preferred_element_type=jnp.float32)
        deg = deg_ref[0, :, 0:1] + deg_ref[1, :, 0:1] + 1.0
        dinv = lax.rsqrt(deg)
        hs_ref[...] = h * dinv
        di_ref[...] = jnp.broadcast_to(dinv, (_N, _H))

    return pl.pallas_call(
        body,
        out_shape=[
            jax.ShapeDtypeStruct((_N, _H), jnp.float32),
            jax.ShapeDtypeStruct((_N, _H), jnp.float32),
        ],
    )(x, W1, degp)


def _tc_mid(S1, hs, dinv16, b1):
    def body(s_ref, hs_ref, di_ref, b_ref, o_ref):
        t = (s_ref[0] + s_ref[1] + hs_ref[...]) * di_ref[...]
        o_ref[...] = jnp.maximum(t + b_ref[...], 0.0) * di_ref[...]

    return pl.pallas_call(
        body,
        out_shape=jax.ShapeDtypeStruct((_N, _H), jnp.float32),
    )(S1, hs, dinv16, b1)


def _tc_out(S2, a, dinv16, W2, b2):
    def body(s_ref, a_ref, di_ref, w_ref, b_ref, o_ref):
        t = (s_ref[0] + s_ref[1] + a_ref[...]) * di_ref[...]
        z = jnp.dot(t, w_ref[...], preferred_element_type=jnp.float32) + b_ref[...]
        m = jnp.max(z, axis=1, keepdims=True)
        e = z - m
        lse = jnp.log(jnp.sum(jnp.exp(e), axis=1, keepdims=True))
        o_ref[...] = e - lse

    return pl.pallas_call(
        body,
        out_shape=jax.ShapeDtypeStruct((_N, _C), jnp.float32),
    )(S2, a, dinv16, W2, b2)


def kernel(x, edge_index, W1, b1, W2, b2):
    src3 = edge_index[0].reshape(_NW, _CH, _B)
    dst3 = edge_index[1].reshape(_NW, _CH, _B)
    ones_rows = jnp.ones((_B, _DW), jnp.float32)
    zeros_d = jnp.zeros((_N, _DW), jnp.float32)
    zeros_h = jnp.zeros((_N, _H), jnp.float32)

    degp = _sc_degree(dst3, ones_rows, zeros_d)
    hs, dinv16 = _tc_pre(x, W1, degp)
    S1 = _sc_scatter16(hs, src3, dst3, zeros_h)
    S2, a = _sc_scatter16_mid(S1, hs, dinv16, b1.reshape(1, _H),
                              src3, dst3, zeros_h)
    return _tc_out(S2, a, dinv16, W2, b2.reshape(1, _C))


# speedup vs baseline: 1.0767x; 1.0002x over previous
import functools

import jax
import jax.numpy as jnp
from jax import lax
from jax.experimental import pallas as pl
from jax.experimental.pallas import tpu as pltpu
from jax.experimental.pallas import tpu_sc as plsc

_N, _E, _FIN, _H, _C = 10000, 320000, 128, 16, 40
_NC, _NS = 2, 16
_NW = _NC * _NS
_EPT = _E // _NW
_B = 100
_CH = _EPT // _B
_K = 10
_G = _CH // _K
_GG = _G // 2
_RPC = _N // _NS
_DW = 16

_SC_PARAMS = pltpu.CompilerParams(use_tc_tiling_on_sc=False)


def _sc_degree(dst3, ones_rows, zeros_d):
    mesh = plsc.VectorSubcoreMesh(core_axis_name="c", subcore_axis_name="s")

    @functools.partial(
        pl.kernel,
        out_type=jax.ShapeDtypeStruct((_NC, _N, _DW), jnp.float32),
        mesh=mesh,
        compiler_params=_SC_PARAMS,
        scratch_types=[
            pltpu.VMEM((_CH, _B), jnp.int32),
            pltpu.VMEM((_B, _DW), jnp.float32),
            pltpu.VMEM_SHARED((_N, _DW), jnp.float32),
            pltpu.SemaphoreType.DMA,
        ],
    )
    def k(dst_hbm, ones_hbm, zeros_hbm, out_hbm, didx, ones_v, acc, ssem):
        c = lax.axis_index("c")
        s = lax.axis_index("s")
        w = s * _NC + c
        pltpu.sync_copy(dst_hbm.at[w], didx)
        pltpu.sync_copy(ones_hbm, ones_v)

        pltpu.sync_copy(zeros_hbm.at[pl.ds(s * _RPC, _RPC)],
                        acc.at[pl.ds(s * _RPC, _RPC)])

        plsc.subcore_barrier()

        def body(g, carry):
            base = g * _K
            for b in range(_K):
                pltpu.async_copy(ones_v, acc.at[didx.at[base + b]], ssem,
                                 add=True)

            @pl.when(g > 0)
            def _():
                for b in range(_K):
                    pltpu.make_async_copy(
                        zeros_hbm.at[pl.ds(0, _B)], ones_v, ssem).wait()

            return carry

        lax.fori_loop(0, _G, body, 0)
        for b in range(_K):
            pltpu.make_async_copy(zeros_hbm.at[pl.ds(0, _B)], ones_v,
                                  ssem).wait()
        plsc.subcore_barrier()

        pltpu.sync_copy(acc.at[pl.ds(s * _RPC, _RPC)],
                        out_hbm.at[c, pl.ds(s * _RPC, _RPC)])

    return k(dst3, ones_rows, zeros_d)


def _sc_scatter16(table, src3, dst3, zeros_h):
    mesh = plsc.VectorSubcoreMesh(core_axis_name="c", subcore_axis_name="s")

    @functools.partial(
        pl.kernel,
        out_type=jax.ShapeDtypeStruct((_NC, _N, _H), jnp.float32),
        mesh=mesh,
        compiler_params=_SC_PARAMS,
        scratch_types=[
            pltpu.VMEM((_CH, _B), jnp.int32),
            pltpu.VMEM((_CH, _B), jnp.int32),
            pltpu.VMEM((2, _K, _B, _H), jnp.float32),
            pltpu.VMEM_SHARED((_N, _H), jnp.float32),
            pltpu.VMEM_SHARED((_N, _H), jnp.float32),
            pltpu.SemaphoreType.DMA,
            pltpu.SemaphoreType.DMA,
        ],
    )
    def k(tab_hbm, src_hbm, dst_hbm, zeros_hbm, out_hbm,
          sidx, didx, rows, acc, tshared, gsem, ssem):
        c = lax.axis_index("c")
        s = lax.axis_index("s")
        w = s * _NC + c
        pltpu.sync_copy(src_hbm.at[w], sidx)
        pltpu.sync_copy(dst_hbm.at[w], didx)

        pltpu.sync_copy(zeros_hbm.at[pl.ds(s * _RPC, _RPC)],
                        acc.at[pl.ds(s * _RPC, _RPC)])
        pltpu.sync_copy(tab_hbm.at[pl.ds(s * _RPC, _RPC)],
                        tshared.at[pl.ds(s * _RPC, _RPC)])

        plsc.subcore_barrier()

        def fire_gathers(base, buf):
            for b in range(_K):
                pltpu.async_copy(tshared.at[sidx.at[base + b]],
                                 rows.at[buf, b], gsem)

        def fire_scatters(base, buf):
            for b in range(_K):
                pltpu.async_copy(rows.at[buf, b], acc.at[didx.at[base + b]],
                                 ssem, add=True)

        def drain(sem):
            for b in range(_K):
                pltpu.make_async_copy(zeros_hbm.at[pl.ds(0, _B)],
                                      rows.at[0, b], sem).wait()

        fire_gathers(0, 0)

        def body(gg, carry):
            g0 = 2 * gg * _K

            drain(gsem)
            fire_scatters(g0, 0)

            @pl.when(gg > 0)
            def _():
                drain(ssem)

            fire_gathers(g0 + _K, 1)
            drain(gsem)
            fire_scatters(g0 + _K, 1)
            drain(ssem)

            @pl.when(gg + 1 < _GG)
            def _():
                fire_gathers(g0 + 2 * _K, 0)

            return carry

        lax.fori_loop(0, _GG, body, 0)
        drain(ssem)
        plsc.subcore_barrier()

        pltpu.sync_copy(acc.at[pl.ds(s * _RPC, _RPC)],
                        out_hbm.at[c, pl.ds(s * _RPC, _RPC)])

    return k(table, src3, dst3, zeros_h)


def _sc_scatter16_mid(S1p, hs, dinv16, b1row, src3, dst3, zeros_h):
    mesh = plsc.VectorSubcoreMesh(core_axis_name="c", subcore_axis_name="s")

    @functools.partial(
        pl.kernel,
        out_type=[
            jax.ShapeDtypeStruct((_NC, _N, _H), jnp.float32),
            jax.ShapeDtypeStruct((_N, _H), jnp.float32),
        ],
        mesh=mesh,
        compiler_params=_SC_PARAMS,
        scratch_types=[
            pltpu.VMEM((_CH, _B), jnp.int32),
            pltpu.VMEM((_CH, _B), jnp.int32),
            pltpu.VMEM((2, _K, _B, _H), jnp.float32),
            pltpu.VMEM((_RPC, _H), jnp.float32),
            pltpu.VMEM((_RPC, _H), jnp.float32),
            pltpu.VMEM((_RPC, _H), jnp.float32),
            pltpu.VMEM((_RPC, _H), jnp.float32),
            pltpu.VMEM((_RPC, _H), jnp.float32),
            pltpu.VMEM((1, _H), jnp.float32),
            pltpu.VMEM_SHARED((_N, _H), jnp.float32),
            pltpu.VMEM_SHARED((_N, _H), jnp.float32),
            pltpu.SemaphoreType.DMA,
            pltpu.SemaphoreType.DMA,
        ],
    )
    def k(s1_hbm, hs_hbm, di_hbm, b1_hbm, src_hbm, dst_hbm, zeros_hbm,
          out_hbm, a_hbm, sidx, didx, rows,
          cs1a, cs1b, chs, cdi, ca, cb1, acc, tshared, gsem, ssem):
        c = lax.axis_index("c")
        s = lax.axis_index("s")
        w = s * _NC + c
        pltpu.sync_copy(src_hbm.at[w], sidx)
        pltpu.sync_copy(dst_hbm.at[w], didx)

        pltpu.sync_copy(zeros_hbm.at[pl.ds(s * _RPC, _RPC)],
                        acc.at[pl.ds(s * _RPC, _RPC)])

        base = s * _RPC
        pltpu.sync_copy(s1_hbm.at[0, pl.ds(base, _RPC)], cs1a)
        pltpu.sync_copy(s1_hbm.at[1, pl.ds(base, _RPC)], cs1b)
        pltpu.sync_copy(hs_hbm.at[pl.ds(base, _RPC)], chs)
        pltpu.sync_copy(di_hbm.at[pl.ds(base, _RPC)], cdi)
        pltpu.sync_copy(b1_hbm, cb1)

        @plsc.parallel_loop(0, _RPC, unroll=4)
        def _(i):
            di = cdi[i]
            t = (cs1a[i] + cs1b[i] + chs[i]) * di
            ca[i] = jnp.maximum(t + cb1[0], 0.0) * di
        pltpu.sync_copy(ca, tshared.at[pl.ds(base, _RPC)])

        @pl.when(c == 0)
        def _():
            pltpu.sync_copy(ca, a_hbm.at[pl.ds(base, _RPC)])

        plsc.subcore_barrier()

        def fire_gathers(gbase, buf):
            for b in range(_K):
                pltpu.async_copy(tshared.at[sidx.at[gbase + b]],
                                 rows.at[buf, b], gsem)

        def fire_scatters(gbase, buf):
            for b in range(_K):
                pltpu.async_copy(rows.at[buf, b], acc.at[didx.at[gbase + b]],
                                 ssem, add=True)

        def drain(sem):
            for b in range(_K):
                pltpu.make_async_copy(zeros_hbm.at[pl.ds(0, _B)],
                                      rows.at[0, b], sem).wait()

        fire_gathers(0, 0)

        def body(gg, carry):
            g0 = 2 * gg * _K

            drain(gsem)
            fire_scatters(g0, 0)

            @pl.when(gg > 0)
            def _():
                drain(ssem)

            fire_gathers(g0 + _K, 1)
            drain(gsem)
            fire_scatters(g0 + _K, 1)
            drain(ssem)

            @pl.when(gg + 1 < _GG)
            def _():
                fire_gathers(g0 + 2 * _K, 0)

            return carry

        lax.fori_loop(0, _GG, body, 0)
        drain(ssem)
        plsc.subcore_barrier()

        pltpu.sync_copy(acc.at[pl.ds(s * _RPC, _RPC)],
                        out_hbm.at[c, pl.ds(s * _RPC, _RPC)])

    return k(S1p, hs, dinv16, b1row, src3, dst3, zeros_h)


def _tc_pre(x, W1, degp):
    def body(x_ref, w_ref, deg_ref, hs_ref, di_ref):
        h = jnp.dot(x_ref[...], w_ref[...], preferred_element_type=jnp.float32)
        deg = deg_ref[0, :, 0:1] + deg_ref[1, :, 0:1] + 1.0
        dinv = lax.rsqrt(deg)
        hs_ref[...] = h * dinv
        di_ref[...] = jnp.broadcast_to(dinv, (_N, _H))

    return pl.pallas_call(
        body,
        out_shape=[
            jax.ShapeDtypeStruct((_N, _H), jnp.float32),
            jax.ShapeDtypeStruct((_N, _H), jnp.float32),
        ],
    )(x, W1, degp)


def _tc_mid(S1, hs, dinv16, b1):
    def body(s_ref, hs_ref, di_ref, b_ref, o_ref):
        t = (s_ref[0] + s_ref[1] + hs_ref[...]) * di_ref[...]
        o_ref[...] = jnp.maximum(t + b_ref[...], 0.0) * di_ref[...]

    return pl.pallas_call(
        body,
        out_shape=jax.ShapeDtypeStruct((_N, _H), jnp.float32),
    )(S1, hs, dinv16, b1)


def _tc_out(S2, a, dinv16, W2, b2):
    def body(s_ref, a_ref, di_ref, w_ref, b_ref, o_ref):
        t = (s_ref[0] + s_ref[1] + a_ref[...]) * di_ref[...]
        z = jnp.dot(t, w_ref[...], preferred_element_type=jnp.float32) + b_ref[...]
        m = jnp.max(z, axis=1, keepdims=True)
        e = z - m
        lse = jnp.log(jnp.sum(jnp.exp(e), axis=1, keepdims=True))
        o_ref[...] = e - lse

    return pl.pallas_call(
        body,
        out_shape=jax.ShapeDtypeStruct((_N, _C), jnp.float32),
    )(S2, a, dinv16, W2, b2)


def kernel(x, edge_index, W1, b1, W2, b2):
    src3 = edge_index[0].reshape(_NW, _CH, _B)
    dst3 = edge_index[1].reshape(_NW, _CH, _B)
    ones_rows = jnp.ones((_B, _DW), jnp.float32)
    zeros_d = jnp.zeros((_N, _DW), jnp.float32)
    zeros_h = jnp.zeros((_N, _H), jnp.float32)

    degp = _sc_degree(dst3, ones_rows, zeros_d)
    hs, dinv16 = _tc_pre(x, W1, degp)
    S1 = _sc_scatter16(hs, src3, dst3, zeros_h)
    S2, a = _sc_scatter16_mid(S1, hs, dinv16, b1.reshape(1, _H),
                              src3, dst3, zeros_h)
    return _tc_out(S2, a, dinv16, W2, b2.reshape(1, _C))
